# Initial kernel scaffold; baseline (speedup 1.0000x reference)
import jax
import jax.numpy as jnp
from jax.experimental import pallas as pl


def kernel(h_lig, h_rec, x_lig, x_rec, edge_index_ll, edge_index_rl, W1e_ll, b1e_ll, W2e_ll, b2e_ll, W1c_ll, b1c_ll, W2c_ll, b2c_ll, W1e_rl, b1e_rl, W2e_rl, b2e_rl, W1c_rl, b1c_rl, W2c_rl, b2c_rl, Wn1, bn1, Wn2, bn2):
    raise NotImplementedError("write your pallas kernel here")


# speedup vs baseline: 2.3857x; 2.3857x over previous
import functools

import jax
import jax.numpy as jnp
from jax import lax
from jax.experimental import pallas as pl
from jax.experimental.pallas import tpu as pltpu
from jax.experimental.pallas import tpu_sc as plsc

N_LIG, N_REC, D, H = 10000, 40000, 128, 128
NC, NS = 2, 16
NW = NC * NS
CB = 128
XW = 16
CHUNKS_LL = 1280
CHUNKS_RL = 3136
EP_LL = CHUNKS_LL * CB
EP_RL = CHUNKS_RL * CB
PW_LL = CHUNKS_LL // NW
PW_RL = CHUNKS_RL // NW
RPS = 632
ACC_ROWS = RPS * NS

_sc_mesh = plsc.VectorSubcoreMesh(core_axis_name="c", subcore_axis_name="s")
_sc_params = pltpu.CompilerParams(use_tc_tiling_on_sc=False)


def _silu(x):
    return x / (1.0 + jnp.exp(-x))


@functools.partial(
    pl.kernel,
    out_type=(
        jax.ShapeDtypeStruct((EP_LL, D), jnp.float32),
        jax.ShapeDtypeStruct((EP_LL, D), jnp.float32),
        jax.ShapeDtypeStruct((EP_LL, XW), jnp.float32),
        jax.ShapeDtypeStruct((EP_LL, XW), jnp.float32),
        jax.ShapeDtypeStruct((EP_RL, D), jnp.float32),
        jax.ShapeDtypeStruct((EP_RL, D), jnp.float32),
        jax.ShapeDtypeStruct((EP_RL, XW), jnp.float32),
        jax.ShapeDtypeStruct((EP_RL, XW), jnp.float32),
    ),
    mesh=_sc_mesh,
    compiler_params=_sc_params,
    scratch_types=[
        pltpu.VMEM((PW_RL, CB), jnp.int32),
        pltpu.VMEM((PW_RL, CB), jnp.int32),
        pltpu.VMEM((CB, D), jnp.float32),
        pltpu.VMEM((CB, D), jnp.float32),
        pltpu.VMEM((CB, XW), jnp.float32),
        pltpu.VMEM((CB, XW), jnp.float32),
        pltpu.SemaphoreType.DMA,
    ],
)
def _gather_kernel(h_lig, h_rec, xl, xr,
                   src_ll, dst_ll, src_rl, dst_rl,
                   hs_ll, hd_ll, xs_ll, xd_ll,
                   hs_rl, hd_rl, xs_rl, xd_rl,
                   idx_s, idx_d, hs_v, hd_v, xs_v, xd_v, sem):
    wid = lax.axis_index("s") * NC + lax.axis_index("c")

    def run(pw, src3, dst3, h_src_t, x_src_t, hs_o, hd_o, xs_o, xd_o):
        base = wid * pw
        pltpu.sync_copy(src3.at[wid], idx_s.at[pl.ds(0, pw)])
        pltpu.sync_copy(dst3.at[wid], idx_d.at[pl.ds(0, pw)])

        def body(j, carry):
            row = base + j
            cs = pltpu.async_copy(h_src_t.at[idx_s.at[j]], hs_v, sem)
            cd = pltpu.async_copy(h_lig.at[idx_d.at[j]], hd_v, sem)
            cxs = pltpu.async_copy(x_src_t.at[idx_s.at[j]], xs_v, sem)
            cxd = pltpu.async_copy(xl.at[idx_d.at[j]], xd_v, sem)
            cs.wait()
            cd.wait()
            cxs.wait()
            cxd.wait()
            pltpu.sync_copy(hs_v, hs_o.at[pl.ds(row * CB, CB)])
            pltpu.sync_copy(hd_v, hd_o.at[pl.ds(row * CB, CB)])
            pltpu.sync_copy(xs_v, xs_o.at[pl.ds(row * CB, CB)])
            pltpu.sync_copy(xd_v, xd_o.at[pl.ds(row * CB, CB)])
            return carry

        lax.fori_loop(0, pw, body, 0)

    run(PW_LL, src_ll, dst_ll, h_lig, xl, hs_ll, hd_ll, xs_ll, xd_ll)
    run(PW_RL, src_rl, dst_rl, h_rec, xr, hs_rl, hd_rl, xs_rl, xd_rl)


_EB = 512


def _edge_mlp_body(hs_ref, hd_ref, xs_ref, xd_ref,
                   w1a, w1b, w1r, b1, w2, b2,
                   v1a, v1b, v1r, c1, w2c, c2,
                   mh_ref, mx_ref):
    hs = hs_ref[...]
    hd = hd_ref[...]
    diff = xs_ref[...] - xd_ref[...]
    d2 = jnp.sum(diff * diff, axis=1, keepdims=True)
    dij = jnp.sqrt(d2)
    xn = diff / (dij + 1e-9)
    pre = (jnp.dot(hs, w1a[...], preferred_element_type=jnp.float32)
           + jnp.dot(hd, w1b[...], preferred_element_type=jnp.float32)
           + dij * w1r[...] + b1[...])
    e1 = _silu(pre)
    mh = _silu(jnp.dot(e1, w2[...], preferred_element_type=jnp.float32) + b2[...])
    prec = (jnp.dot(hs, v1a[...], preferred_element_type=jnp.float32)
            + jnp.dot(hd, v1b[...], preferred_element_type=jnp.float32)
            + dij * v1r[...] + c1[...])
    e1c = _silu(prec)
    cc = _silu(jnp.dot(e1c, w2c[...], preferred_element_type=jnp.float32) + c2[...])
    mh_ref[...] = mh
    mx_ref[...] = cc[:, 0:1] * xn


def _edge_mlp(ep, hs, hd, xs, xd, ws):
    grid = ep // _EB
    eb = lambda i: (i, 0)
    wb = lambda i: (0, 0)
    return pl.pallas_call(
        _edge_mlp_body,
        grid=(grid,),
        in_specs=[
            pl.BlockSpec((_EB, D), eb), pl.BlockSpec((_EB, D), eb),
            pl.BlockSpec((_EB, XW), eb), pl.BlockSpec((_EB, XW), eb),
            pl.BlockSpec((D, H), wb), pl.BlockSpec((D, H), wb),
            pl.BlockSpec((1, H), wb), pl.BlockSpec((1, H), wb),
            pl.BlockSpec((H, H), wb), pl.BlockSpec((1, H), wb),
            pl.BlockSpec((D, H), wb), pl.BlockSpec((D, H), wb),
            pl.BlockSpec((1, H), wb), pl.BlockSpec((1, H), wb),
            pl.BlockSpec((H, XW), wb), pl.BlockSpec((1, XW), wb),
        ],
        out_specs=[
            pl.BlockSpec((_EB, H), eb),
            pl.BlockSpec((_EB, XW), eb),
        ],
        out_shape=[
            jax.ShapeDtypeStruct((ep, H), jnp.float32),
            jax.ShapeDtypeStruct((ep, XW), jnp.float32),
        ],
    )(hs, hd, xs, xd, *ws)


@functools.partial(
    pl.kernel,
    out_type=(
        jax.ShapeDtypeStruct((NC, ACC_ROWS, D), jnp.float32),
        jax.ShapeDtypeStruct((NC, ACC_ROWS, XW), jnp.float32),
    ),
    mesh=_sc_mesh,
    compiler_params=_sc_params,
    scratch_types=[
        pltpu.VMEM((PW_RL, CB), jnp.int32),
        pltpu.VMEM((CB, D), jnp.float32),
        pltpu.VMEM((CB, XW), jnp.float32),
        pltpu.VMEM_SHARED((ACC_ROWS, D), jnp.float32),
        pltpu.VMEM_SHARED((ACC_ROWS, XW), jnp.float32),
        pltpu.SemaphoreType.DMA,
    ],
)
def _scatter_kernel(mh_ll, mx_ll, mh_rl, mx_rl, dsts_ll, dsts_rl, zh, zx,
                    part_h, part_x, idx_d, mh_v, mx_v, acc_h, acc_x, sem):
    cid = lax.axis_index("c")
    sid = lax.axis_index("s")
    wid = sid * NC + cid
    r0 = sid * RPS
    pltpu.sync_copy(zh.at[pl.ds(r0, RPS)], acc_h.at[pl.ds(r0, RPS)])
    pltpu.sync_copy(zx.at[pl.ds(r0, RPS)], acc_x.at[pl.ds(r0, RPS)])
    plsc.subcore_barrier()

    def run(pw, dst3, mh_hbm, mx_hbm):
        base = wid * pw
        pltpu.sync_copy(dst3.at[wid], idx_d.at[pl.ds(0, pw)])

        def body(j, carry):
            row = base + j
            pltpu.sync_copy(mh_hbm.at[pl.ds(row * CB, CB)], mh_v)
            pltpu.sync_copy(mx_hbm.at[pl.ds(row * CB, CB)], mx_v)
            pltpu.sync_copy(mh_v, acc_h.at[idx_d.at[j]], add=True)
            pltpu.sync_copy(mx_v, acc_x.at[idx_d.at[j]], add=True)
            return carry

        lax.fori_loop(0, pw, body, 0)

    run(PW_LL, dsts_ll, mh_ll, mx_ll)
    run(PW_RL, dsts_rl, mh_rl, mx_rl)
    plsc.subcore_barrier()
    pltpu.sync_copy(acc_h.at[pl.ds(r0, RPS)], part_h.at[cid, pl.ds(r0, RPS)])
    pltpu.sync_copy(acc_x.at[pl.ds(r0, RPS)], part_x.at[cid, pl.ds(r0, RPS)])


_NB = 1000


def _node_body(h_ref, ph0, ph1, xl_ref, px0, px1,
               wn1a, wn1b, bn1, wn2, bn2, ho_ref, xo_ref):
    h = h_ref[...]
    hn = ph0[...] + ph1[...]
    pre = (jnp.dot(h, wn1a[...], preferred_element_type=jnp.float32)
           + jnp.dot(hn, wn1b[...], preferred_element_type=jnp.float32)
           + bn1[...])
    m = jnp.dot(_silu(pre), wn2[...], preferred_element_type=jnp.float32) + bn2[...]
    ho_ref[...] = h + m
    xo_ref[...] = xl_ref[...] + px0[...] + px1[...]


def _node_call(h_lig, ph0, ph1, xl, px0, px1, wn1a, wn1b, bn1, wn2, bn2):
    nb = lambda i: (i, 0)
    wb = lambda i: (0, 0)
    return pl.pallas_call(
        _node_body,
        grid=(N_LIG // _NB,),
        in_specs=[
            pl.BlockSpec((_NB, D), nb), pl.BlockSpec((_NB, D), nb),
            pl.BlockSpec((_NB, D), nb),
            pl.BlockSpec((_NB, XW), nb), pl.BlockSpec((_NB, XW), nb),
            pl.BlockSpec((_NB, XW), nb),
            pl.BlockSpec((D, H), wb), pl.BlockSpec((D, H), wb),
            pl.BlockSpec((1, H), wb), pl.BlockSpec((H, D), wb),
            pl.BlockSpec((1, D), wb),
        ],
        out_specs=[
            pl.BlockSpec((_NB, D), nb),
            pl.BlockSpec((_NB, XW), nb),
        ],
        out_shape=[
            jax.ShapeDtypeStruct((N_LIG, D), jnp.float32),
            jax.ShapeDtypeStruct((N_LIG, XW), jnp.float32),
        ],
    )(h_lig, ph0, ph1, xl, px0, px1, wn1a, wn1b, bn1, wn2, bn2)


def _prep_idx(ei, ep):
    e = ei.shape[1]
    src = jnp.pad(ei[0], (0, ep - e))
    dst_g = jnp.pad(ei[1], (0, ep - e))
    dst_s = jnp.pad(ei[1], (0, ep - e), constant_values=N_LIG)
    return (src.reshape(NW, -1, CB), dst_g.reshape(NW, -1, CB),
            dst_s.reshape(NW, -1, CB))


def kernel(h_lig, h_rec, x_lig, x_rec, edge_index_ll, edge_index_rl,
           W1e_ll, b1e_ll, W2e_ll, b2e_ll, W1c_ll, b1c_ll, W2c_ll, b2c_ll,
           W1e_rl, b1e_rl, W2e_rl, b2e_rl, W1c_rl, b1c_rl, W2c_rl, b2c_rl,
           Wn1, bn1, Wn2, bn2):
    xl = jnp.pad(x_lig, ((0, 0), (0, XW - 3)))
    xr = jnp.pad(x_rec, ((0, 0), (0, XW - 3)))
    src_ll, dstg_ll, dsts_ll = _prep_idx(edge_index_ll, EP_LL)
    src_rl, dstg_rl, dsts_rl = _prep_idx(edge_index_rl, EP_RL)

    (hs_ll, hd_ll, xs_ll, xd_ll,
     hs_rl, hd_rl, xs_rl, xd_rl) = _gather_kernel(
        h_lig, h_rec, xl, xr, src_ll, dstg_ll, src_rl, dstg_rl)

    def ws(W1e, b1e, W2e, b2e, W1c, b1c, W2c, b2c):
        return (W1e[:D], W1e[D:2 * D], W1e[2 * D:], b1e.reshape(1, H),
                W2e, b2e.reshape(1, H),
                W1c[:D], W1c[D:2 * D], W1c[2 * D:], b1c.reshape(1, H),
                jnp.pad(W2c, ((0, 0), (0, XW - 1))),
                jnp.pad(b2c, (0, XW - 1)).reshape(1, XW))

    mh_ll, mx_ll = _edge_mlp(EP_LL, hs_ll, hd_ll, xs_ll, xd_ll,
                             ws(W1e_ll, b1e_ll, W2e_ll, b2e_ll,
                                W1c_ll, b1c_ll, W2c_ll, b2c_ll))
    mh_rl, mx_rl = _edge_mlp(EP_RL, hs_rl, hd_rl, xs_rl, xd_rl,
                             ws(W1e_rl, b1e_rl, W2e_rl, b2e_rl,
                                W1c_rl, b1c_rl, W2c_rl, b2c_rl))

    zh = jnp.zeros((ACC_ROWS, D), jnp.float32)
    zx = jnp.zeros((ACC_ROWS, XW), jnp.float32)
    part_h, part_x = _scatter_kernel(mh_ll, mx_ll, mh_rl, mx_rl,
                                     dsts_ll, dsts_rl, zh, zx)

    h_out, xo = _node_call(h_lig, part_h[0], part_h[1], xl,
                           part_x[0], part_x[1],
                           Wn1[:D], Wn1[D:], bn1.reshape(1, H),
                           Wn2, bn2.reshape(1, D))
    return (h_out, h_rec, xo[:, :3], x_rec)
